# Initial kernel scaffold; baseline (speedup 1.0000x reference)
import jax
import jax.numpy as jnp
from jax.experimental import pallas as pl


def kernel(motion_input, codebook):
    raise NotImplementedError("write your pallas kernel here")


# speedup vs baseline: 1.4475x; 1.4475x over previous
import functools

import jax
import jax.numpy as jnp
from jax.experimental import pallas as pl
from jax.experimental.pallas import tpu as pltpu


def _vq_block_kernel(x_ref, cb_ref, q_ref, idx_ref, loss_ref, *, blk_n: int, k: int):
    i = pl.program_id(0)
    x = x_ref[:]
    cb = cb_ref[:]
    xn = jnp.sum(x * x, axis=1, keepdims=True)
    cn = jnp.sum(cb * cb, axis=1)
    xc = jax.lax.dot_general(
        x, cb, (((1,), (1,)), ((), ())), preferred_element_type=jnp.float32
    )
    dist = -(xn - 2.0 * xc + cn[None, :])
    idx = jnp.argmax(dist, axis=-1).astype(jnp.int32)
    onehot = (
        jax.lax.broadcasted_iota(jnp.int32, (blk_n, k), 1) == idx[:, None]
    ).astype(jnp.float32)
    q = jax.lax.dot_general(
        onehot, cb, (((1,), (0,)), ((), ())), preferred_element_type=jnp.float32
    )
    q_ref[:] = q
    idx_ref[0, 0, :] = idx
    diff = q - x
    part = jnp.sum(diff * diff)

    @pl.when(i == 0)
    def _():
        loss_ref[0, 0] = part

    @pl.when(i > 0)
    def _():
        loss_ref[0, 0] += part


def kernel(motion_input, codebook):
    b, t, d = motion_input.shape
    k = codebook.shape[0]
    n = b * t
    blk_n = 512
    nb = n // blk_n
    flat = motion_input.reshape(n, d)

    q, idx, loss_sum = pl.pallas_call(
        functools.partial(_vq_block_kernel, blk_n=blk_n, k=k),
        grid=(nb,),
        in_specs=[
            pl.BlockSpec((blk_n, d), lambda i: (i, 0)),
            pl.BlockSpec((k, d), lambda i: (0, 0)),
        ],
        out_specs=[
            pl.BlockSpec((blk_n, d), lambda i: (i, 0)),
            pl.BlockSpec((1, 1, blk_n), lambda i: (i, 0, 0)),
            pl.BlockSpec(memory_space=pltpu.SMEM),
        ],
        out_shape=[
            jax.ShapeDtypeStruct((n, d), jnp.float32),
            jax.ShapeDtypeStruct((nb, 1, blk_n), jnp.int32),
            jax.ShapeDtypeStruct((1, 1), jnp.float32),
        ],
    )(flat, codebook)

    quantize = q.reshape(b, t, d)
    embed_ind = idx.reshape(b, t)
    loss = loss_sum[0, 0] / jnp.float32(n * d)
    return (quantize, embed_ind, loss)
